# Initial kernel scaffold; baseline (speedup 1.0000x reference)
import jax
import jax.numpy as jnp
from jax.experimental import pallas as pl


def kernel(x, P, log_tau, log_blend):
    raise NotImplementedError("write your pallas kernel here")


# speedup vs baseline: 3.4997x; 3.4997x over previous
import functools
import math

import jax
import jax.numpy as jnp
from jax.experimental import pallas as pl
from jax.experimental.pallas import tpu as pltpu

_N = 8192
_D = 768
_K = 512
_BN = 1024
_SQ2OPI = math.sqrt(2.0 / math.pi)


def _row_normalize(v, eps):
    n = jnp.sqrt(jnp.sum(v * v, axis=-1, keepdims=True))
    return v / jnp.maximum(n, eps)


def _assign_kernel(x_ref, p_ref, assign_ref, sums_ref, counts_ref):
    i = pl.program_id(0)
    xb = x_ref[...]
    p_norm = _row_normalize(p_ref[...], 1e-12)
    sims = jax.lax.dot_general(xb, p_norm, (((1,), (1,)), ((), ())),
                               preferred_element_type=jnp.float32)
    a = jnp.argmax(sims, axis=-1).astype(jnp.int32)
    assign_ref[...] = a.reshape(1, 1, _BN)
    onehot_t = (jax.lax.broadcasted_iota(jnp.int32, (_K, _BN), 0)
                == a[None, :]).astype(jnp.float32)
    part_sums = jax.lax.dot_general(onehot_t, xb, (((1,), (0,)), ((), ())),
                                    preferred_element_type=jnp.float32)
    part_counts = jnp.sum(onehot_t, axis=1).reshape(1, _K)

    @pl.when(i == 0)
    def _init():
        sums_ref[...] = part_sums
        counts_ref[...] = part_counts

    @pl.when(i != 0)
    def _acc():
        sums_ref[...] += part_sums
        counts_ref[...] += part_counts


def _update_kernel(p_ref, sums_ref, counts_ref, pn2_ref):
    p0 = p_ref[...]
    counts = counts_ref[...].reshape(_K, 1)
    sums = sums_ref[...]
    centroids = jnp.where(counts > 0, sums / jnp.maximum(counts, 1.0), p0)
    new_p = _row_normalize(centroids, 1e-12)
    p_upd = 0.999 * p0 + 0.001 * new_p
    pn2_ref[...] = _row_normalize(p_upd, 1e-08)


def _out_kernel(x_ref, pn2_ref, lt_ref, lb_ref, out_ref):
    xb = x_ref[...]
    pn2 = pn2_ref[...]
    s2 = jax.lax.dot_general(xb, pn2, (((1,), (1,)), ((), ())),
                             preferred_element_type=jnp.float32)
    rowmax = jnp.max(s2, axis=-1)
    xnorm = jnp.sqrt(jnp.sum(xb * xb, axis=-1))
    m = rowmax / jnp.maximum(xnorm, 1e-08)
    m = jnp.clip(m, -1.0, 1.0)
    dists = jnp.clip(1.0 - m, 0.0, 2.0)
    tau = jnp.exp(lt_ref[0, 0])
    alpha = jax.nn.sigmoid(lb_ref[0, 0])
    novelty = 1.0 - jnp.exp(-tau * dists)
    scale = jnp.clip(1.0 - alpha + alpha * novelty, 0.1, 10.0)[:, None]
    y = xb * scale
    out_ref[...] = 0.5 * y * (1.0 + jnp.tanh(_SQ2OPI * (y + 0.044715 * y**3)))


@jax.jit
def _run(x2d, P, log_tau, log_blend):
    nblk = _N // _BN
    assign3, sums, counts = pl.pallas_call(
        _assign_kernel,
        grid=(nblk,),
        in_specs=[
            pl.BlockSpec((_BN, _D), lambda i: (i, 0)),
            pl.BlockSpec((_K, _D), lambda i: (0, 0)),
        ],
        out_specs=[
            pl.BlockSpec((1, 1, _BN), lambda i: (i, 0, 0)),
            pl.BlockSpec((_K, _D), lambda i: (0, 0)),
            pl.BlockSpec((1, _K), lambda i: (0, 0)),
        ],
        out_shape=[
            jax.ShapeDtypeStruct((nblk, 1, _BN), jnp.int32),
            jax.ShapeDtypeStruct((_K, _D), jnp.float32),
            jax.ShapeDtypeStruct((1, _K), jnp.float32),
        ],
    )(x2d, P)
    del assign3

    pn2 = pl.pallas_call(
        _update_kernel,
        in_specs=[
            pl.BlockSpec((_K, _D), lambda: (0, 0)),
            pl.BlockSpec((_K, _D), lambda: (0, 0)),
            pl.BlockSpec((1, _K), lambda: (0, 0)),
        ],
        out_specs=pl.BlockSpec((_K, _D), lambda: (0, 0)),
        out_shape=jax.ShapeDtypeStruct((_K, _D), jnp.float32),
    )(P, sums, counts)

    out2d = pl.pallas_call(
        _out_kernel,
        grid=(nblk,),
        in_specs=[
            pl.BlockSpec((_BN, _D), lambda i: (i, 0)),
            pl.BlockSpec((_K, _D), lambda i: (0, 0)),
            pl.BlockSpec(memory_space=pltpu.SMEM),
            pl.BlockSpec(memory_space=pltpu.SMEM),
        ],
        out_specs=pl.BlockSpec((_BN, _D), lambda i: (i, 0)),
        out_shape=jax.ShapeDtypeStruct((_N, _D), jnp.float32),
    )(x2d, pn2, log_tau, log_blend)
    return out2d


def kernel(x, P, log_tau, log_blend):
    B, T, D = x.shape
    x2d = x.reshape(-1, D)
    lt = jnp.reshape(log_tau, (1, 1))
    lb = jnp.reshape(log_blend, (1, 1))
    out2d = _run(x2d, P, lt, lb)
    return out2d.reshape(B, T, D)
